# Initial kernel scaffold; baseline (speedup 1.0000x reference)
import jax
import jax.numpy as jnp
from jax.experimental import pallas as pl


def kernel(odm_loc_data, odm_conf_data, refined_anchors, ignore_flags_refined_anchor):
    raise NotImplementedError("write your pallas kernel here")


# speedup vs baseline: 1.3549x; 1.3549x over previous
import jax
import jax.numpy as jnp
from jax.experimental import pallas as pl
from jax.experimental.pallas import tpu as pltpu

NUM_CLASSES = 21
TOP_K = 200
CONF_THRESH = 0.01
NMS_THRESH = 0.45
VAR0, VAR1 = 0.1, 0.2

N = 20000
BN = 2000


def _prep_body(conf_ref, loc_ref, anc_ref, scores_ref, boxes_ref):
    c = conf_ref[...]
    m = jnp.max(c, axis=-1, keepdims=True)
    e = jnp.exp(c - m)
    scores_ref[...] = e / jnp.sum(e, axis=-1, keepdims=True)

    loc = loc_ref[...]
    anc = anc_ref[...]
    xy = anc[:, :2] + loc[:, :2] * VAR0 * anc[:, 2:]
    wh = anc[:, 2:] * jnp.exp(loc[:, 2:] * VAR1)
    mins = xy - wh * 0.5
    maxs = mins + wh
    boxes_ref[...] = jnp.concatenate([mins, maxs], axis=1)


def _prep(conf2d, loc2d, anc2d):
    nrows = conf2d.shape[0]
    grid = (nrows // BN,)
    return pl.pallas_call(
        _prep_body,
        grid=grid,
        in_specs=[
            pl.BlockSpec((BN, NUM_CLASSES), lambda i: (i, 0)),
            pl.BlockSpec((BN, 4), lambda i: (i, 0)),
            pl.BlockSpec((BN, 4), lambda i: (i, 0)),
        ],
        out_specs=[
            pl.BlockSpec((BN, NUM_CLASSES), lambda i: (i, 0)),
            pl.BlockSpec((BN, 4), lambda i: (i, 0)),
        ],
        out_shape=[
            jax.ShapeDtypeStruct((nrows, NUM_CLASSES), jnp.float32),
            jax.ShapeDtypeStruct((nrows, 4), jnp.float32),
        ],
    )(conf2d, loc2d, anc2d)


def _iou_matrix(b):
    x1, y1, x2, y2 = b[:, 0], b[:, 1], b[:, 2], b[:, 3]
    area = (x2 - x1) * (y2 - y1)
    ix1 = jnp.maximum(x1[:, None], x1[None, :])
    iy1 = jnp.maximum(y1[:, None], y1[None, :])
    ix2 = jnp.minimum(x2[:, None], x2[None, :])
    iy2 = jnp.minimum(y2[:, None], y2[None, :])
    inter = jnp.clip(ix2 - ix1, 0.0) * jnp.clip(iy2 - iy1, 0.0)
    union = area[:, None] + area[None, :] - inter
    return inter / jnp.maximum(union, 1e-12)


def _nms_single(boxes, scores, valid):
    K = TOP_K
    masked = jnp.where(valid, scores, -jnp.inf)
    top_s, top_i = jax.lax.top_k(masked, K)
    cboxes = boxes[top_i]
    cvalid = top_s > CONF_THRESH
    iou = _iou_matrix(cboxes)
    idxs = jnp.arange(K)

    def body(i, keep):
        sup = jnp.any((iou[i] > NMS_THRESH) & keep & (idxs < i))
        return keep.at[i].set(cvalid[i] & jnp.logical_not(sup))

    keep = jax.lax.fori_loop(0, K, body, jnp.zeros((K,), dtype=bool))
    sort_key = (1 - keep.astype(jnp.int32)) * K + idxs
    order = jnp.argsort(sort_key)
    keep_s = keep[order]
    det = jnp.concatenate([top_s[order][:, None], cboxes[order]], axis=1)
    return jnp.where(keep_s[:, None], det, 0.0)


@jax.jit
def _run(odm_loc, odm_conf, anchors, ignore):
    B = odm_loc.shape[0]
    scores2d, boxes2d = _prep(
        odm_conf.reshape(B * N, NUM_CLASSES),
        odm_loc.reshape(B * N, 4),
        anchors.reshape(B * N, 4),
    )
    scores = scores2d.reshape(B, N, NUM_CLASSES)
    boxes = boxes2d.reshape(B, N, 4)

    def per_image(boxes_i, sc_i, ign_i):
        flag = ign_i < 1

        def per_class(sc_c):
            valid = flag & (sc_c > CONF_THRESH)
            return _nms_single(boxes_i, sc_c, valid)

        dets = jax.vmap(per_class, in_axes=1)(sc_i[:, 1:])
        zero = jnp.zeros((1, TOP_K, 5), dets.dtype)
        return jnp.concatenate([zero, dets], axis=0)

    return jax.vmap(per_image)(boxes, scores, ignore)


def kernel(odm_loc_data, odm_conf_data, refined_anchors, ignore_flags_refined_anchor):
    return _run(odm_loc_data, odm_conf_data, refined_anchors, ignore_flags_refined_anchor)


# speedup vs baseline: 2.0682x; 1.5264x over previous
import jax
import jax.numpy as jnp
from jax.experimental import pallas as pl
from jax.experimental.pallas import tpu as pltpu

NUM_CLASSES = 21
TOP_K = 200
CONF_THRESH = 0.01
NMS_THRESH = 0.45
VAR0, VAR1 = 0.1, 0.2

N = 20000
BN = 2000
KPAD = 256


def _prep_body(conf_ref, loc_ref, anc_ref, ign_ref, scores_ref, boxes_ref):
    c = conf_ref[...]
    m = jnp.max(c, axis=-1, keepdims=True)
    e = jnp.exp(c - m)
    s = e / jnp.sum(e, axis=-1, keepdims=True)
    flag = (ign_ref[...] < 1)
    valid = flag & (s > CONF_THRESH)
    scores_ref[...] = jnp.where(valid, s, -1.0)

    loc = loc_ref[...]
    anc = anc_ref[...]
    xy = anc[:, :2] + loc[:, :2] * VAR0 * anc[:, 2:]
    wh = anc[:, 2:] * jnp.exp(loc[:, 2:] * VAR1)
    mins = xy - wh * 0.5
    maxs = mins + wh
    boxes_ref[...] = jnp.concatenate([mins, maxs], axis=1)


def _prep(conf2d, loc2d, anc2d, ign2d):
    nrows = conf2d.shape[0]
    grid = (nrows // BN,)
    return pl.pallas_call(
        _prep_body,
        grid=grid,
        in_specs=[
            pl.BlockSpec((BN, NUM_CLASSES), lambda i: (i, 0)),
            pl.BlockSpec((BN, 4), lambda i: (i, 0)),
            pl.BlockSpec((BN, 4), lambda i: (i, 0)),
            pl.BlockSpec((BN, 1), lambda i: (i, 0)),
        ],
        out_specs=[
            pl.BlockSpec((BN, NUM_CLASSES), lambda i: (i, 0)),
            pl.BlockSpec((BN, 4), lambda i: (i, 0)),
        ],
        out_shape=[
            jax.ShapeDtypeStruct((nrows, NUM_CLASSES), jnp.float32),
            jax.ShapeDtypeStruct((nrows, 4), jnp.float32),
        ],
    )(conf2d, loc2d, anc2d, ign2d)


def _nms_body(s_ref, x1_ref, y1_ref, x2_ref, y2_ref,
              os_ref, ox1_ref, oy1_ref, ox2_ref, oy2_ref):
    R = s_ref.shape[0]
    s = s_ref[...]
    x1 = x1_ref[...]
    y1 = y1_ref[...]
    x2 = x2_ref[...]
    y2 = y2_ref[...]
    lane = jax.lax.broadcasted_iota(jnp.int32, (R, KPAD), 1)
    area = (x2 - x1) * (y2 - y1)

    def _col(v, j):
        return jnp.sum(jnp.where(lane == j, v, 0.0), axis=1, keepdims=True)

    def body(i, keep):
        x1i = _col(x1, i)
        y1i = _col(y1, i)
        x2i = _col(x2, i)
        y2i = _col(y2, i)
        ix1 = jnp.maximum(x1i, x1)
        iy1 = jnp.maximum(y1i, y1)
        ix2 = jnp.minimum(x2i, x2)
        iy2 = jnp.minimum(y2i, y2)
        inter = jnp.clip(ix2 - ix1, 0.0) * jnp.clip(iy2 - iy1, 0.0)
        areai = (x2i - x1i) * (y2i - y1i)
        union = jnp.maximum(areai + area - inter, 1e-12)
        over = inter > NMS_THRESH * union
        sup = jnp.max(jnp.where(over, keep, 0.0), axis=1, keepdims=True)
        vi = _col(s, i) > CONF_THRESH
        newk = jnp.where(vi & (sup == 0.0), 1.0, 0.0)
        return jnp.where(lane == i, newk, keep)

    keep = jax.lax.fori_loop(0, TOP_K, body, jnp.zeros((R, KPAD), jnp.float32))

    kf = keep
    r_i = jax.lax.broadcasted_iota(jnp.int32, (KPAD, KPAD), 0)
    c_i = jax.lax.broadcasted_iota(jnp.int32, (KPAD, KPAD), 1)
    tri = jnp.where(r_i < c_i, 1.0, 0.0)
    pos = jax.lax.dot(kf, tri, preferred_element_type=jnp.float32)

    z = jnp.zeros((R, KPAD), jnp.float32)

    def cstep(j, outs):
        o_s, o_x1, o_y1, o_x2, o_y2 = outs
        kj = _col(kf, j) > 0.0
        pj = _col(pos, j)
        m = kj & (pj == lane.astype(jnp.float32))
        o_s = jnp.where(m, _col(s, j), o_s)
        o_x1 = jnp.where(m, _col(x1, j), o_x1)
        o_y1 = jnp.where(m, _col(y1, j), o_y1)
        o_x2 = jnp.where(m, _col(x2, j), o_x2)
        o_y2 = jnp.where(m, _col(y2, j), o_y2)
        return (o_s, o_x1, o_y1, o_x2, o_y2)

    o_s, o_x1, o_y1, o_x2, o_y2 = jax.lax.fori_loop(
        0, TOP_K, cstep, (z, z, z, z, z))
    os_ref[...] = o_s
    ox1_ref[...] = o_x1
    oy1_ref[...] = o_y1
    ox2_ref[...] = o_x2
    oy2_ref[...] = o_y2


def _nms(s, x1, y1, x2, y2):
    R = s.shape[0]
    outs = [jax.ShapeDtypeStruct((R, KPAD), jnp.float32)] * 5
    return pl.pallas_call(
        _nms_body,
        out_shape=outs,
    )(s, x1, y1, x2, y2)


@jax.jit
def _run(odm_loc, odm_conf, anchors, ignore):
    B = odm_loc.shape[0]
    scores2d, boxes2d = _prep(
        odm_conf.reshape(B * N, NUM_CLASSES),
        odm_loc.reshape(B * N, 4),
        anchors.reshape(B * N, 4),
        ignore.reshape(B * N, 1),
    )
    scores = scores2d.reshape(B, N, NUM_CLASSES)
    boxes = boxes2d.reshape(B, N, 4)

    sc_t = jnp.transpose(scores[:, :, 1:], (0, 2, 1)).reshape(B * (NUM_CLASSES - 1), N)
    top_s, top_i = jax.lax.top_k(sc_t, TOP_K)
    R = B * (NUM_CLASSES - 1)
    img = jnp.arange(R) // (NUM_CLASSES - 1)
    cbox = boxes[img[:, None], top_i]

    pad = ((0, 0), (0, KPAD - TOP_K))
    s_p = jnp.pad(top_s, pad, constant_values=-1.0)
    x1 = jnp.pad(cbox[:, :, 0], pad)
    y1 = jnp.pad(cbox[:, :, 1], pad)
    x2 = jnp.pad(cbox[:, :, 2], pad)
    y2 = jnp.pad(cbox[:, :, 3], pad)

    o_s, o_x1, o_y1, o_x2, o_y2 = _nms(s_p, x1, y1, x2, y2)
    dets = jnp.stack(
        [o_s[:, :TOP_K], o_x1[:, :TOP_K], o_y1[:, :TOP_K],
         o_x2[:, :TOP_K], o_y2[:, :TOP_K]], axis=-1)
    dets = dets.reshape(B, NUM_CLASSES - 1, TOP_K, 5)
    zero = jnp.zeros((B, 1, TOP_K, 5), dets.dtype)
    return jnp.concatenate([zero, dets], axis=1)


def kernel(odm_loc_data, odm_conf_data, refined_anchors, ignore_flags_refined_anchor):
    return _run(odm_loc_data, odm_conf_data, refined_anchors, ignore_flags_refined_anchor)
